# baseline (speedup 1.0000x reference)
import functools

import jax
import jax.numpy as jnp
from jax import lax
from jax.experimental import pallas as pl
from jax.experimental.pallas import tpu as pltpu
from jax.experimental.pallas import tpu_sc as plsc

N_TOKENS = 32768
D_MODEL = 768
NUM_EXPERTS = 8
LANES = 128

N_SC = 4096
N_TC = N_TOKENS - N_SC
BT = 4096
NW = 32
CHUNK = 32
NSUB = N_SC // (NW * CHUNK)
NGRP = CHUNK // 16


def _tc_body(x_ref, wt_ref, gates_ref, idx_ref, probs_ref):
    logits = jnp.dot(x_ref[...], wt_ref[...],
                     preferred_element_type=jnp.float32)
    lt = jnp.transpose(logits)[:NUM_EXPERTS, :]
    row = jax.lax.broadcasted_iota(jnp.int32, lt.shape, 0)
    m = jnp.max(lt, axis=0, keepdims=True)
    e = jnp.exp(lt - m)
    s = jnp.sum(e, axis=0, keepdims=True)
    p = e / s

    p1 = jnp.max(p, axis=0, keepdims=True)
    i1 = jnp.min(jnp.where(p == p1, row, NUM_EXPERTS), axis=0, keepdims=True)
    p_rest = jnp.where(row == i1, jnp.float32(-1.0), p)
    p2 = jnp.max(p_rest, axis=0, keepdims=True)
    i2 = jnp.min(jnp.where(p_rest == p2, row, NUM_EXPERTS), axis=0,
                 keepdims=True)
    denom = p1 + p2
    probs_ref[...] = p
    gates_ref[...] = jnp.concatenate([p1 / denom, p2 / denom], axis=0)
    idx_ref[...] = jnp.concatenate([i1, i2], axis=0)


def _tc_router(x, wt):
    grid = (N_TC // BT,)
    return pl.pallas_call(
        _tc_body,
        grid=grid,
        in_specs=[
            pl.BlockSpec((BT, D_MODEL), lambda i: (i, 0)),
            pl.BlockSpec((D_MODEL, LANES), lambda i: (0, 0)),
        ],
        out_specs=[
            pl.BlockSpec((2, BT), lambda i: (0, i)),
            pl.BlockSpec((2, BT), lambda i: (0, i)),
            pl.BlockSpec((NUM_EXPERTS, BT), lambda i: (0, i)),
        ],
        out_shape=[
            jax.ShapeDtypeStruct((2, N_TC), jnp.float32),
            jax.ShapeDtypeStruct((2, N_TC), jnp.int32),
            jax.ShapeDtypeStruct((NUM_EXPERTS, N_TC), jnp.float32),
        ],
    )(x, wt)


def _round_bf16(v):
    u = plsc.bitcast(v, jnp.uint32)
    r = (u + jnp.uint32(0x7FFF) + ((u >> 16) & jnp.uint32(1))) \
        & jnp.uint32(0xFFFF0000)
    return plsc.bitcast(r, jnp.float32)


def _tree_max(vs):
    while len(vs) > 1:
        vs = [jnp.maximum(a, b) for a, b in zip(vs[::2], vs[1::2])]
    return vs[0]


_sc_mesh = plsc.VectorSubcoreMesh(core_axis_name="c", subcore_axis_name="s")


@functools.partial(
    pl.kernel,
    out_type=(
        jax.ShapeDtypeStruct((2, N_SC), jnp.float32),
        jax.ShapeDtypeStruct((2, N_SC), jnp.int32),
        jax.ShapeDtypeStruct((NUM_EXPERTS, N_SC), jnp.float32),
    ),
    mesh=_sc_mesh,
    compiler_params=pltpu.CompilerParams(use_tc_tiling_on_sc=False, needs_layout_passes=False),
    scratch_types=[
        pltpu.VMEM((CHUNK, D_MODEL + 1), jnp.float32),
        pltpu.VMEM((D_MODEL, 16), jnp.float32),
        pltpu.VMEM((NUM_EXPERTS, CHUNK), jnp.float32),
        pltpu.VMEM((2, CHUNK), jnp.float32),
        pltpu.VMEM((2, CHUNK), jnp.int32),
    ],
)
def _sc_router(x_hbm, wt_hbm, gates_hbm, idx_hbm, probs_hbm,
               xbuf, wtbuf, pbuf, gbuf, ibuf):
    wid = lax.axis_index("s") * 2 + lax.axis_index("c")
    pltpu.sync_copy(wt_hbm, wtbuf)
    iota16 = lax.iota(jnp.int32, 16)
    rows = [iota16 + 16 * g for g in range(NGRP)]
    esplat = [jnp.full((16,), e, jnp.int32) for e in range(NUM_EXPERTS)]
    zero16 = jnp.zeros((16,), jnp.float32)

    for sub in range(NSUB):
        base = wid * (CHUNK * NSUB) + sub * CHUNK
        pltpu.sync_copy(x_hbm.at[pl.ds(base, CHUNK)],
                        xbuf.at[:, pl.ds(0, D_MODEL)])

        def kbody(k, accs):
            wrow = wtbuf[k]
            ws = [wrow.at[esplat[e]].get(mode="promise_in_bounds")
                  for e in range(NUM_EXPERTS)]
            kvec = lax.broadcast(k, (16,))
            out = []
            for g in range(NGRP):
                xg = plsc.load_gather(xbuf, [rows[g], kvec])
                xr = _round_bf16(xg)
                out.append(tuple(accs[g][e] + xr * ws[e]
                                 for e in range(NUM_EXPERTS)))
            return tuple(out)

        init = tuple(tuple(zero16 for _ in range(NUM_EXPERTS))
                     for _ in range(NGRP))
        accs = lax.fori_loop(0, D_MODEL, kbody, init, unroll=4)

        for g in range(NGRP):
            l = list(accs[g])
            m = _tree_max(l)
            ex = [jnp.exp(v - m) for v in l]
            s = ex[0]
            for v in ex[1:]:
                s = s + v
            p = [v / s for v in ex]
            p1 = _tree_max(p)
            i1 = jnp.full((16,), NUM_EXPERTS, jnp.int32)
            for e in range(NUM_EXPERTS - 1, -1, -1):
                i1 = jnp.where(p[e] == p1, esplat[e], i1)
            prest = [jnp.where(esplat[e] == i1, jnp.float32(-1.0), p[e])
                     for e in range(NUM_EXPERTS)]
            p2 = _tree_max(prest)
            i2 = jnp.full((16,), NUM_EXPERTS, jnp.int32)
            for e in range(NUM_EXPERTS - 1, -1, -1):
                i2 = jnp.where(prest[e] == p2, esplat[e], i2)
            denom = p1 + p2
            sl = pl.ds(16 * g, 16)
            for e in range(NUM_EXPERTS):
                pbuf[e, sl] = p[e]
            gbuf[0, sl] = p1 / denom
            gbuf[1, sl] = p2 / denom
            ibuf[0, sl] = i1
            ibuf[1, sl] = i2

        dst = pl.ds(base, CHUNK)
        pltpu.sync_copy(pbuf, probs_hbm.at[:, dst])
        pltpu.sync_copy(gbuf, gates_hbm.at[:, dst])
        pltpu.sync_copy(ibuf, idx_hbm.at[:, dst])


def kernel(x, W_r):
    wt_tc = jnp.pad(W_r.T, ((0, 0), (0, LANES - NUM_EXPERTS)))
    wt_sc = jnp.pad(W_r.T.astype(jnp.bfloat16).astype(jnp.float32),
                    ((0, 0), (0, 16 - NUM_EXPERTS)))
    x_sc = lax.slice(x, (N_TC, 0), (N_TOKENS, D_MODEL))
    g_sc, i_sc, p_sc = _sc_router(x_sc, wt_sc)
    g_tc, i_tc, p_tc = _tc_router(x, wt_tc)
    gates = jnp.concatenate([g_tc, g_sc], axis=1).T
    idx = jnp.concatenate([i_tc, i_sc], axis=1).T
    probs = jnp.concatenate([p_tc, p_sc], axis=1).T
    return gates, idx, probs


# speedup vs baseline: 3.0189x; 3.0189x over previous
import jax
import jax.numpy as jnp
from jax.experimental import pallas as pl

N_TOKENS = 32768
D_MODEL = 768
NUM_EXPERTS = 8
LANES = 128
BT = 4096


def _router_body(x_ref, wt_ref, gates_ref, idx_ref, probs_ref):
    logits = jnp.dot(x_ref[...], wt_ref[...],
                     preferred_element_type=jnp.float32)
    lt = jnp.transpose(logits)[:NUM_EXPERTS, :]
    row = jax.lax.broadcasted_iota(jnp.int32, lt.shape, 0)
    m = jnp.max(lt, axis=0, keepdims=True)
    e = jnp.exp(lt - m)
    s = jnp.sum(e, axis=0, keepdims=True)
    p = e / s

    p1 = jnp.max(p, axis=0, keepdims=True)
    i1 = jnp.min(jnp.where(p == p1, row, NUM_EXPERTS), axis=0, keepdims=True)
    p_rest = jnp.where(row == i1, jnp.float32(-1.0), p)
    p2 = jnp.max(p_rest, axis=0, keepdims=True)
    i2 = jnp.min(jnp.where(p_rest == p2, row, NUM_EXPERTS), axis=0,
                 keepdims=True)
    denom = p1 + p2
    probs_ref[...] = p
    gates_ref[...] = jnp.concatenate([p1 / denom, p2 / denom], axis=0)
    idx_ref[...] = jnp.concatenate([i1, i2], axis=0)


def kernel(x, W_r):
    wt = jnp.pad(W_r.T, ((0, 0), (0, LANES - NUM_EXPERTS)))
    grid = (N_TOKENS // BT,)
    gates_t, idx_t, probs_t = pl.pallas_call(
        _router_body,
        grid=grid,
        in_specs=[
            pl.BlockSpec((BT, D_MODEL), lambda i: (i, 0)),
            pl.BlockSpec((D_MODEL, LANES), lambda i: (0, 0)),
        ],
        out_specs=[
            pl.BlockSpec((2, BT), lambda i: (0, i)),
            pl.BlockSpec((2, BT), lambda i: (0, i)),
            pl.BlockSpec((NUM_EXPERTS, BT), lambda i: (0, i)),
        ],
        out_shape=[
            jax.ShapeDtypeStruct((2, N_TOKENS), jnp.float32),
            jax.ShapeDtypeStruct((2, N_TOKENS), jnp.int32),
            jax.ShapeDtypeStruct((NUM_EXPERTS, N_TOKENS), jnp.float32),
        ],
    )(x, wt)
    return gates_t.T, idx_t.T, probs_t.T
